# baseline (speedup 1.0000x reference)
import jax
import jax.numpy as jnp
import numpy as np
from jax import lax
from jax.experimental import pallas as pl
from jax.experimental.pallas import tpu as pltpu
from jax.experimental.pallas import tpu_sc as plsc

_D = 128
_OUT = 12

_O3, _O1, _O2 = 0, 12, 16


def _body(pack_hbm, tab_hbm, out_hbm, smem, sem):
    pltpu.sync_copy(pack_hbm, smem)

    def val(k):
        return smem[2 * k] + smem[2 * k + 1]

    descs = []
    for k in range(_OUT):
        i = k // 3
        j = k % 3
        idx = val(_O1 + i) * 4096 + val(_O2 + j) * 64 + val(_O3 + k)
        descs.append(pltpu.async_copy(
            tab_hbm.at[pl.ds(idx, 1)], out_hbm.at[pl.ds(np.int32(k), 1)], sem))
    for d in descs:
        d.wait()


def _gather12(pack, tab):
    mesh = plsc.ScalarSubcoreMesh(axis_name="c", num_cores=1)
    f = pl.kernel(
        _body,
        mesh=mesh,
        out_type=jax.ShapeDtypeStruct((_OUT, _D), jnp.float32),
        scratch_types=[
            pltpu.SMEM((40,), jnp.int32),
            pltpu.SemaphoreType.DMA,
        ],
    )
    return f(pack, tab)


def kernel(x, index1, index2, index3):
    tab = x.reshape(-1, _D)
    cat = jnp.concatenate([
        index3.reshape(-1), index1.reshape(-1), index2.reshape(-1),
        jnp.zeros((1,), index3.dtype),
    ])
    pack = lax.bitcast_convert_type(cat, jnp.int32).reshape(-1)
    out = _gather12(pack, tab)
    return out.reshape(4, 3, _D)


# speedup vs baseline: 1.1289x; 1.1289x over previous
import jax
import jax.numpy as jnp
import numpy as np
from jax import lax
from jax.experimental import pallas as pl
from jax.experimental.pallas import tpu as pltpu
from jax.experimental.pallas import tpu_sc as plsc

_D = 128
_OUT = 12

_O3, _O1, _O2 = 0, 12, 16


def _body(pack_hbm, tab_hbm, out_hbm, smem, sem):
    pltpu.sync_copy(pack_hbm, smem)
    descs = []
    for k in range(_OUT):
        i = k // 3
        j = k % 3
        idx = smem[_O1 + i] * 4096 + smem[_O2 + j] * 64 + smem[_O3 + k]
        descs.append(pltpu.async_copy(
            tab_hbm.at[pl.ds(idx, 1)],
            out_hbm.at[np.int32(i), pl.ds(np.int32(j), 1)], sem))
    for d in descs:
        d.wait()


def _gather12(pack, tab):
    mesh = plsc.ScalarSubcoreMesh(axis_name="c", num_cores=1)
    f = pl.kernel(
        _body,
        mesh=mesh,
        out_type=jax.ShapeDtypeStruct((4, 3, _D), jnp.float32),
        scratch_types=[
            pltpu.SMEM((32,), jnp.int32),
            pltpu.SemaphoreType.DMA,
        ],
    )
    return f(pack, tab)


def kernel(x, index1, index2, index3):
    tab = x.reshape(-1, _D)
    zpad = jnp.zeros((4,), index3.dtype)
    cat = jnp.concatenate([
        index3.reshape(-1), index1.reshape(-1), index2.reshape(-1),
        zpad, zpad, zpad, jnp.zeros((1,), index3.dtype),
    ])
    pack = cat.astype(jnp.int32)
    return _gather12(pack, tab)
